# initial kernel scaffold (unmeasured)
import functools

import jax
import jax.numpy as jnp
from jax import lax
from jax.experimental import pallas as pl
from jax.experimental.pallas import tpu as pltpu

B, SQ, H, D = 4, 32, 8, 128
SKV_SHARD = 4096
K_CHUNK = 512
N_CHUNKS = SKV_SHARD // K_CHUNK
SCALE = D ** -0.5

DeviceIdType = getattr(pl, "DeviceIdType", None) or pltpu.DeviceIdType
semaphore_signal = getattr(pl, "semaphore_signal", None) or pltpu.semaphore_signal
semaphore_wait = getattr(pl, "semaphore_wait", None) or pltpu.semaphore_wait


def kernel(Q, K, V):
    def body(q_ref, k_ref, v_ref, out_ref,
             acc_ref, m_ref, l_ref,
             peer_o, peer_m, peer_l,
             send_sems, recv_sems):
        step = pl.program_id(0)
        my_x = lax.axis_index("x")
        my_y = lax.axis_index("y")
        my_z = lax.axis_index("z")
        peer = (1 - my_x, my_y, my_z)

        @pl.when(step == 0)
        def _init():
            barrier_sem = pltpu.get_barrier_semaphore()
            semaphore_signal(barrier_sem, inc=1, device_id=peer,
                             device_id_type=DeviceIdType.MESH)
            semaphore_wait(barrier_sem, 1)
            m_ref[...] = jnp.full((B, H, SQ, 1), -1e30, jnp.float32)
            l_ref[...] = jnp.zeros((B, H, SQ, 1), jnp.float32)
            acc_ref[...] = jnp.zeros((B, H, SQ, D), jnp.float32)

        for b in range(B):
            for h in range(H):
                q = q_ref[b, :, h, :]
                k = k_ref[b, :, h, :]
                v = v_ref[b, :, h, :]
                s = lax.dot_general(
                    q, k, (((1,), (1,)), ((), ())),
                    preferred_element_type=jnp.float32,
                ) * SCALE
                m_old = m_ref[b, h]
                m_new = jnp.maximum(m_old, jnp.max(s, axis=1, keepdims=True))
                alpha = jnp.exp(m_old - m_new)
                p = jnp.exp(s - m_new)
                pv = lax.dot_general(
                    p, v, (((1,), (0,)), ((), ())),
                    preferred_element_type=jnp.float32,
                )
                l_ref[b, h] = l_ref[b, h] * alpha + jnp.sum(s * 0 + p, axis=1, keepdims=True)
                acc_ref[b, h] = acc_ref[b, h] * alpha + pv
                m_ref[b, h] = m_new

        @pl.when(step == N_CHUNKS - 1)
        def _exchange_and_merge():
            rdmas = []
            for i, (src, dst) in enumerate(
                [(acc_ref, peer_o), (m_ref, peer_m), (l_ref, peer_l)]
            ):
                rdma = pltpu.make_async_remote_copy(
                    src_ref=src, dst_ref=dst,
                    send_sem=send_sems.at[i], recv_sem=recv_sems.at[i],
                    device_id=peer, device_id_type=DeviceIdType.MESH,
                )
                rdma.start()
                rdmas.append(rdma)
            for rdma in rdmas:
                rdma.wait()

            m1, l1, o1 = m_ref[...], l_ref[...], acc_ref[...]
            m2, l2, o2 = peer_m[...], peer_l[...], peer_o[...]
            m = jnp.maximum(m1, m2)
            a1 = jnp.exp(m1 - m)
            a2 = jnp.exp(m2 - m)
            l = l1 * a1 + l2 * a2
            o = (o1 * a1 + o2 * a2) / l
            for b in range(B):
                for h in range(H):
                    out_ref[b, :, h, :] = o[b, h]

    grid = (N_CHUNKS,)
    return pl.pallas_call(
        body,
        grid=grid,
        out_shape=jax.ShapeDtypeStruct((B, SQ, H, D), jnp.float32),
        in_specs=[
            pl.BlockSpec((B, SQ, H, D), lambda i: (0, 0, 0, 0),
                         memory_space=pltpu.VMEM),
            pl.BlockSpec((B, K_CHUNK, H, D), lambda i: (0, i, 0, 0),
                         memory_space=pltpu.VMEM),
            pl.BlockSpec((B, K_CHUNK, H, D), lambda i: (0, i, 0, 0),
                         memory_space=pltpu.VMEM),
        ],
        out_specs=pl.BlockSpec((B, SQ, H, D), lambda i: (0, 0, 0, 0),
                               memory_space=pltpu.VMEM),
        scratch_shapes=[
            pltpu.VMEM((B, H, SQ, D), jnp.float32),
            pltpu.VMEM((B, H, SQ, 1), jnp.float32),
            pltpu.VMEM((B, H, SQ, 1), jnp.float32),
            pltpu.VMEM((B, H, SQ, D), jnp.float32),
            pltpu.VMEM((B, H, SQ, 1), jnp.float32),
            pltpu.VMEM((B, H, SQ, 1), jnp.float32),
            pltpu.SemaphoreType.DMA((3,)),
            pltpu.SemaphoreType.DMA((3,)),
        ],
        compiler_params=pltpu.CompilerParams(
            collective_id=0,
            dimension_semantics=("arbitrary",),
        ),
    )(Q, K, V)


# baseline (device time: 140730 ns/iter reference)
import functools

import jax
import jax.numpy as jnp
from jax import lax
from jax.experimental import pallas as pl
from jax.experimental.pallas import tpu as pltpu

B, SQ, H, D = 4, 32, 8, 128
SKV_SHARD = 4096
K_CHUNK = 512
N_CHUNKS = SKV_SHARD // K_CHUNK
SCALE = D ** -0.5

DeviceIdType = getattr(pl, "DeviceIdType", None) or pltpu.DeviceIdType
semaphore_signal = getattr(pl, "semaphore_signal", None) or pltpu.semaphore_signal
semaphore_wait = getattr(pl, "semaphore_wait", None) or pltpu.semaphore_wait


def kernel(Q, K, V):
    def body(q_ref, k_ref, v_ref, out_ref,
             acc_ref, m_ref, l_ref,
             peer_o, peer_m, peer_l,
             send_sems, recv_sems):
        step = pl.program_id(0)
        my_x = lax.axis_index("x")
        my_y = lax.axis_index("y")
        my_z = lax.axis_index("z")
        peer = (1 - my_x, my_y, my_z)

        @pl.when(step == 0)
        def _init():
            barrier_sem = pltpu.get_barrier_semaphore()
            semaphore_signal(barrier_sem, inc=1, device_id=peer,
                             device_id_type=DeviceIdType.MESH)
            semaphore_wait(barrier_sem, 1)
            m_ref[...] = jnp.full((B, H, SQ, 1), -1e30, jnp.float32)
            l_ref[...] = jnp.zeros((B, H, SQ, 1), jnp.float32)
            acc_ref[...] = jnp.zeros((B, H, SQ, D), jnp.float32)

        for b in range(B):
            for h in range(H):
                q = q_ref[b, :, h, :]
                k = k_ref[b, :, h, :]
                v = v_ref[b, :, h, :]
                s = lax.dot_general(
                    q, k, (((1,), (1,)), ((), ())),
                    preferred_element_type=jnp.float32,
                ) * SCALE
                m_old = m_ref[b, h]
                m_new = jnp.maximum(m_old, jnp.max(s, axis=1, keepdims=True))
                alpha = jnp.exp(m_old - m_new)
                p = jnp.exp(s - m_new)
                pv = lax.dot_general(
                    p, v, (((1,), (0,)), ((), ())),
                    preferred_element_type=jnp.float32,
                )
                l_ref[b, h] = l_ref[b, h] * alpha + jnp.sum(p, axis=1, keepdims=True)
                acc_ref[b, h] = acc_ref[b, h] * alpha + pv
                m_ref[b, h] = m_new

        @pl.when(step == N_CHUNKS - 1)
        def _exchange_and_merge():
            rdmas = []
            for i, (src, dst) in enumerate(
                [(acc_ref, peer_o), (m_ref, peer_m), (l_ref, peer_l)]
            ):
                rdma = pltpu.make_async_remote_copy(
                    src_ref=src, dst_ref=dst,
                    send_sem=send_sems.at[i], recv_sem=recv_sems.at[i],
                    device_id=peer, device_id_type=DeviceIdType.MESH,
                )
                rdma.start()
                rdmas.append(rdma)
            for rdma in rdmas:
                rdma.wait()

            m1, l1, o1 = m_ref[...], l_ref[...], acc_ref[...]
            m2, l2, o2 = peer_m[...], peer_l[...], peer_o[...]
            m = jnp.maximum(m1, m2)
            a1 = jnp.exp(m1 - m)
            a2 = jnp.exp(m2 - m)
            l = l1 * a1 + l2 * a2
            o = (o1 * a1 + o2 * a2) / l
            for b in range(B):
                for h in range(H):
                    out_ref[b, :, h, :] = o[b, h]

    grid = (N_CHUNKS,)
    return pl.pallas_call(
        body,
        grid=grid,
        out_shape=jax.ShapeDtypeStruct((B, SQ, H, D), jnp.float32),
        in_specs=[
            pl.BlockSpec((B, SQ, H, D), lambda i: (0, 0, 0, 0),
                         memory_space=pltpu.VMEM),
            pl.BlockSpec((B, K_CHUNK, H, D), lambda i: (0, i, 0, 0),
                         memory_space=pltpu.VMEM),
            pl.BlockSpec((B, K_CHUNK, H, D), lambda i: (0, i, 0, 0),
                         memory_space=pltpu.VMEM),
        ],
        out_specs=pl.BlockSpec((B, SQ, H, D), lambda i: (0, 0, 0, 0),
                               memory_space=pltpu.VMEM),
        scratch_shapes=[
            pltpu.VMEM((B, H, SQ, D), jnp.float32),
            pltpu.VMEM((B, H, SQ, 1), jnp.float32),
            pltpu.VMEM((B, H, SQ, 1), jnp.float32),
            pltpu.VMEM((B, H, SQ, D), jnp.float32),
            pltpu.VMEM((B, H, SQ, 1), jnp.float32),
            pltpu.VMEM((B, H, SQ, 1), jnp.float32),
            pltpu.SemaphoreType.DMA((3,)),
            pltpu.SemaphoreType.DMA((3,)),
        ],
        compiler_params=pltpu.CompilerParams(
            collective_id=0,
            dimension_semantics=("arbitrary",),
            vmem_limit_bytes=100 * 1024 * 1024,
        ),
    )(Q, K, V)
